# Initial kernel scaffold; baseline (speedup 1.0000x reference)
import jax
import jax.numpy as jnp
from jax.experimental import pallas as pl


def kernel(x, edge_index, edge_attr, weight, w_ih, w_hh, b_ih, b_hh):
    raise NotImplementedError("write your pallas kernel here")


# speedup vs baseline: 4.3897x; 4.3897x over previous
import functools

import jax
import jax.numpy as jnp
from jax import lax
from jax.experimental import pallas as pl
from jax.experimental.pallas import tpu as pltpu
from jax.experimental.pallas import tpu_sc as plsc

N = 10000
D = 128
NC = 2
NS = 16
NW = NC * NS
CH = 128
LANES = 16

ROW_BLK = 2000
NP = 10240


def _mm_body(x_ref, w_ref, o_ref):
    o_ref[...] = jnp.dot(x_ref[...], w_ref[...],
                         preferred_element_type=jnp.float32)


def _gru_body(p_ref, x_ref, wih_ref, whh_ref, bih_ref, bhh_ref, o_ref):
    agg = p_ref[0] + p_ref[1]
    h = x_ref[...]
    gi = jnp.dot(agg, wih_ref[...],
                 preferred_element_type=jnp.float32) + bih_ref[...]
    gh = jnp.dot(h, whh_ref[...],
                 preferred_element_type=jnp.float32) + bhh_ref[...]
    r = jax.nn.sigmoid(gi[:, :D] + gh[:, :D])
    z = jax.nn.sigmoid(gi[:, D:2 * D] + gh[:, D:2 * D])
    n = jnp.tanh(gi[:, 2 * D:] + r * gh[:, 2 * D:])
    o_ref[...] = (1.0 - z) * n + z * h


def _sc_body(m_hbm, src_hbm, dst_hbm, attr_hbm, out_hbm,
             src_v, dst_v, attr_v, rows_v, agg_sh, sem):
    nch = src_hbm.shape[1]
    rpw = NP // NS
    cid = lax.axis_index("c")
    sid = lax.axis_index("s")
    wid = sid * NC + cid

    def _zrow(i, carry):
        for c in range(D // LANES):
            rows_v[i, pl.ds(c * LANES, LANES)] = jnp.zeros(
                (LANES,), jnp.float32)
        return carry
    lax.fori_loop(0, CH, _zrow, 0)

    base = sid * rpw
    nfull = rpw // CH
    for t in range(nfull):
        pltpu.sync_copy(rows_v, agg_sh.at[pl.ds(base + t * CH, CH)])
    plsc.subcore_barrier()

    pltpu.sync_copy(src_hbm.at[wid], src_v)
    pltpu.sync_copy(dst_hbm.at[wid], dst_v)
    pltpu.sync_copy(attr_hbm.at[wid], attr_v)

    def _chunk(j, carry):
        pltpu.async_copy(m_hbm.at[src_v.at[j]], rows_v, sem).wait()

        def _group(g, c2):
            a16 = attr_v[pl.ds(j * CH + g * LANES, LANES)]
            for e in range(LANES):
                s = a16[e]
                row = g * LANES + e
                for c in range(D // LANES):
                    sl = pl.ds(c * LANES, LANES)
                    rows_v[row, sl] = rows_v[row, sl] * s
            return c2
        lax.fori_loop(0, CH // LANES, _group, 0)

        pltpu.sync_copy(rows_v, agg_sh.at[dst_v.at[j]], add=True)
        return carry
    lax.fori_loop(0, nch, _chunk, 0)

    plsc.subcore_barrier()
    pltpu.sync_copy(agg_sh.at[pl.ds(base, rpw)],
                    out_hbm.at[cid, pl.ds(base, rpw)])


def _make_sc_call(nch):
    mesh = plsc.VectorSubcoreMesh(core_axis_name="c", subcore_axis_name="s")
    return pl.kernel(
        _sc_body,
        mesh=mesh,
        out_type=jax.ShapeDtypeStruct((NC, NP, D), jnp.float32),
        scratch_types=[
            pltpu.VMEM((nch, CH), jnp.int32),
            pltpu.VMEM((nch, CH), jnp.int32),
            pltpu.VMEM((nch * CH,), jnp.float32),
            pltpu.VMEM((CH, D), jnp.float32),
            pltpu.VMEM_SHARED((NP, D), jnp.float32),
            pltpu.SemaphoreType.DMA,
        ],
    )


def kernel(x, edge_index, edge_attr, weight, w_ih, w_hh, b_ih, b_hh):
    E = edge_attr.shape[0]
    src = edge_index[0].astype(jnp.int32)
    dst = edge_index[1].astype(jnp.int32)
    attr = edge_attr.astype(jnp.float32)

    quantum = NW * CH
    e_pad = ((E + quantum - 1) // quantum) * quantum
    pad = e_pad - E
    if pad:
        src = jnp.concatenate([src, jnp.zeros((pad,), jnp.int32)])
        dst = jnp.concatenate([dst, jnp.zeros((pad,), jnp.int32)])
        attr = jnp.concatenate([attr, jnp.zeros((pad,), jnp.float32)])
    nch = e_pad // quantum
    src3 = src.reshape(NW, nch, CH)
    dst3 = dst.reshape(NW, nch, CH)
    attr2 = attr.reshape(NW, nch * CH)

    grid = N // ROW_BLK
    m = pl.pallas_call(
        _mm_body,
        grid=(grid,),
        in_specs=[
            pl.BlockSpec((ROW_BLK, D), lambda i: (i, 0)),
            pl.BlockSpec((D, D), lambda i: (0, 0)),
        ],
        out_specs=pl.BlockSpec((ROW_BLK, D), lambda i: (i, 0)),
        out_shape=jax.ShapeDtypeStruct((N, D), jnp.float32),
    )(x, weight[0])

    partials = _make_sc_call(nch)(m, src3, dst3, attr2)

    wih_t = w_ih.T
    whh_t = w_hh.T
    bih = b_ih.reshape(1, 3 * D)
    bhh = b_hh.reshape(1, 3 * D)
    h = pl.pallas_call(
        _gru_body,
        grid=(grid,),
        in_specs=[
            pl.BlockSpec((NC, ROW_BLK, D), lambda i: (0, i, 0)),
            pl.BlockSpec((ROW_BLK, D), lambda i: (i, 0)),
            pl.BlockSpec((D, 3 * D), lambda i: (0, 0)),
            pl.BlockSpec((D, 3 * D), lambda i: (0, 0)),
            pl.BlockSpec((1, 3 * D), lambda i: (0, 0)),
            pl.BlockSpec((1, 3 * D), lambda i: (0, 0)),
        ],
        out_specs=pl.BlockSpec((ROW_BLK, D), lambda i: (i, 0)),
        out_shape=jax.ShapeDtypeStruct((N, D), jnp.float32),
    )(partials, x, wih_t, whh_t, bih, bhh)
    return h
